# baseline (speedup 1.0000x reference)
import functools

import numpy as np
import jax
import jax.numpy as jnp
from jax.experimental import pallas as pl
from jax.experimental.pallas import tpu as pltpu

KS = 32
STRIDE = 16
BS = 32
TOPN = 16
NINIT = 2
WIN = 512
NEG = -1e30


def _masked_softmax(s, mask):
    sm = jnp.where(mask, s, NEG)
    m = jnp.max(sm, axis=-1, keepdims=True)
    e = jnp.where(mask, jnp.exp(sm - m), 0.0)
    den = jnp.maximum(jnp.sum(e, axis=-1, keepdims=True), 1e-30)
    return e / den


def _nsa_kernel(q_ref, k_ref, v_ref, w_ref, m_ref, e_ref, t_ref, c_ref, wm_ref,
                o_ref, sel_scr, *, BQ, TS, H, D, J):
    i = pl.program_id(1)
    t0 = i * BQ
    BQH = BQ * H
    scale = D ** -0.5

    q = q_ref[0].reshape(BQH, D)
    ks = k_ref[0]
    vs = v_ref[0]

    nch = TS // STRIDE
    c16k = jnp.mean(ks.reshape(nch, STRIDE, D), axis=1)
    c16v = jnp.mean(vs.reshape(nch, STRIDE, D), axis=1)
    cmpk = (c16k + jnp.concatenate([c16k[1:], c16k[-1:]], axis=0)) * 0.5
    cmpv = (c16v + jnp.concatenate([c16v[1:], c16v[-1:]], axis=0)) * 0.5

    sc = jax.lax.dot_general(q, cmpk, (((1,), (1,)), ((), ())),
                             preferred_element_type=jnp.float32) * scale
    sc3 = sc.reshape(BQ, H, nch)
    trow = t0 + jax.lax.broadcasted_iota(jnp.int32, (BQ, 1, 1), 0)
    cidx = jax.lax.broadcasted_iota(jnp.int32, (1, 1, nch), 2)
    cmask = (cidx * STRIDE + (KS - 1)) <= trow
    p3 = _masked_softmax(sc3, cmask)
    cmp_o = jnp.dot(p3.reshape(BQH, nch), cmpv,
                    preferred_element_type=jnp.float32)

    p_sum = jnp.sum(p3, axis=1)
    p_slc = jnp.dot(p_sum, m_ref[...],
                    preferred_element_type=jnp.float32)

    tq = t0 + jax.lax.broadcasted_iota(jnp.int32, (BQ, 1), 0)
    jidx = jax.lax.broadcasted_iota(jnp.int32, (1, J), 1)
    blk_valid = (jidx * BS) <= tq
    cur = tq // BS
    forced = ((jidx < NINIT) | (jidx == cur)) & blk_valid
    score = jnp.where(blk_valid, p_slc + forced.astype(jnp.float32) * 1024.0, -1.0)

    JJ = J * J
    a = jnp.dot(score, e_ref[...][:, :JJ], precision=jax.lax.Precision.HIGHEST,
                preferred_element_type=jnp.float32)
    bb = jnp.dot(score, t_ref[...], precision=jax.lax.Precision.HIGHEST,
                 preferred_element_type=jnp.float32)
    pidx = jax.lax.broadcasted_iota(jnp.int32, (1, JJ), 1)
    lane_lt = (pidx % J) < (pidx // J)
    beats = (bb > a) | ((bb == a) & lane_lt)
    rank = jax.lax.dot_general(beats.astype(jnp.float32), e_ref[...][:, :JJ],
                               (((1,), (1,)), ((), ())),
                               preferred_element_type=jnp.float32)
    sel = (rank < min(TOPN, J)) & blk_valid

    selx = jnp.dot(sel.astype(jnp.float32), e_ref[...],
                   preferred_element_type=jnp.float32)

    sel_scr[...] = selx * c_ref[0]
    CK = BQ

    def chunk_body(c, carry):
        acc_slc, acc_swa, den_slc, den_swa = carry
        kc = k_ref[0, pl.ds(c * CK, CK), :]
        vc = v_ref[0, pl.ds(c * CK, CK), :]
        s_c = jax.lax.dot_general(q, kc, (((1,), (1,)), ((), ())),
                                  preferred_element_type=jnp.float32) * scale
        e_c = jnp.exp(s_c).reshape(BQ, H, CK)
        sm = sel_scr[:, pl.ds(c * CK, CK)][:, None, :]
        wm = wm_ref[0, :, pl.ds(c * CK, CK)][:, None, :]
        e1 = e_c * sm
        e2 = e_c * wm
        den_slc = den_slc + jnp.sum(e1, axis=-1, keepdims=True)
        den_swa = den_swa + jnp.sum(e2, axis=-1, keepdims=True)
        acc_slc = acc_slc + jnp.dot(e1.reshape(BQH, CK), vc,
                                    preferred_element_type=jnp.float32)
        acc_swa = acc_swa + jnp.dot(e2.reshape(BQH, CK), vc,
                                    preferred_element_type=jnp.float32)
        return acc_slc, acc_swa, den_slc, den_swa

    zero_o = jnp.zeros((BQH, D), jnp.float32)
    zero_d = jnp.zeros((BQ, H, 1), jnp.float32)
    acc_slc, acc_swa, den_slc, den_swa = jax.lax.fori_loop(
        0, i + 1, chunk_body, (zero_o, zero_o, zero_d, zero_d))
    slc_o = acc_slc / den_slc.reshape(BQH, 1)
    swa_o = acc_swa / den_swa.reshape(BQH, 1)

    g = jax.nn.sigmoid(w_ref[0])
    out = g[:, 0:1] * cmp_o + g[:, 1:2] * slc_o + g[:, 2:3] * swa_o
    o_ref[...] = out.reshape(1, BQ, H, D)


def kernel(q, k, v, combine_weight, cu_seqlens):
    T, H, D = q.shape
    nseq = cu_seqlens.shape[0] - 1
    TS = T // nseq
    BQ = 128
    J = (TS + BS - 1) // BS
    nch = TS // STRIDE

    C = (TS - KS) // STRIDE + 1
    M_np = np.zeros((nch, J), np.float32)
    for c in range(C):
        s0 = (c * STRIDE) // BS
        s1 = (c * STRIDE + KS - 1) // BS
        M_np[c, s0:s1 + 1] = 1.0
    E_np = np.zeros((J, TS), np.float32)
    for j in range(J):
        E_np[j, j * BS:(j + 1) * BS] = 1.0
    JJ = J * J
    TILE_np = np.zeros((J, JJ), np.float32)
    for jp in range(J):
        TILE_np[jp, jp::J] = 1.0
    NQB = TS // BQ
    tpos = np.arange(TS)
    caus_np = np.zeros((NQB, BQ, TS), np.float32)
    win_np = np.zeros((NQB, BQ, TS), np.float32)
    for i in range(NQB):
        t = (i * BQ + np.arange(BQ))[:, None]
        caus_np[i] = (tpos[None, :] <= t).astype(np.float32)
        win_np[i] = ((tpos[None, :] <= t) & (tpos[None, :] > t - WIN)).astype(np.float32)

    q4 = q.reshape(nseq, TS, H, D)
    k4 = k.reshape(nseq, TS, D)
    v4 = v.reshape(nseq, TS, D)
    w4 = combine_weight.reshape(nseq, TS * H, 3)

    fn = functools.partial(_nsa_kernel, BQ=BQ, TS=TS, H=H, D=D, J=J)
    out = pl.pallas_call(
        fn,
        grid=(nseq, TS // BQ),
        in_specs=[
            pl.BlockSpec((1, BQ, H, D), lambda b, i: (b, i, 0, 0)),
            pl.BlockSpec((1, TS, D), lambda b, i: (b, 0, 0)),
            pl.BlockSpec((1, TS, D), lambda b, i: (b, 0, 0)),
            pl.BlockSpec((1, BQ * H, 3), lambda b, i: (b, i, 0)),
            pl.BlockSpec((nch, J), lambda b, i: (0, 0)),
            pl.BlockSpec((J, TS), lambda b, i: (0, 0)),
            pl.BlockSpec((J, JJ), lambda b, i: (0, 0)),
            pl.BlockSpec((1, BQ, TS), lambda b, i: (i, 0, 0)),
            pl.BlockSpec((1, BQ, TS), lambda b, i: (i, 0, 0)),
        ],
        out_specs=pl.BlockSpec((1, BQ, H, D), lambda b, i: (b, i, 0, 0)),
        out_shape=jax.ShapeDtypeStruct((nseq, TS, H, D), jnp.float32),
        scratch_shapes=[pltpu.VMEM((BQ, TS), jnp.float32)],
    )(q4, k4, v4, w4, jnp.asarray(M_np), jnp.asarray(E_np),
      jnp.asarray(TILE_np), jnp.asarray(caus_np), jnp.asarray(win_np))
    return out.reshape(T, H, D)


# speedup vs baseline: 1.4028x; 1.4028x over previous
import functools

import numpy as np
import jax
import jax.numpy as jnp
from jax.experimental import pallas as pl
from jax.experimental.pallas import tpu as pltpu

KS = 32
STRIDE = 16
BS = 32
TOPN = 16
NINIT = 2
WIN = 512
NEG = -1e30


def _masked_softmax(s, mask):
    sm = jnp.where(mask, s, NEG)
    m = jnp.max(sm, axis=-1, keepdims=True)
    e = jnp.where(mask, jnp.exp(sm - m), 0.0)
    den = jnp.maximum(jnp.sum(e, axis=-1, keepdims=True), 1e-30)
    return e / den


def _nsa_kernel(q_ref, k_ref, v_ref, w_ref, m_ref, e_ref, t_ref, c_ref, wm_ref,
                o_ref, *, BQ, TS, H, D, J):
    i = pl.program_id(1)
    t0 = i * BQ
    BQH = BQ * H
    scale = D ** -0.5

    q = q_ref[0].reshape(BQH, D)
    ks = k_ref[0]
    vs = v_ref[0]

    nch = TS // STRIDE
    c16k = jnp.mean(ks.reshape(nch, STRIDE, D), axis=1)
    c16v = jnp.mean(vs.reshape(nch, STRIDE, D), axis=1)
    cmpk = (c16k + jnp.concatenate([c16k[1:], c16k[-1:]], axis=0)) * 0.5
    cmpv = (c16v + jnp.concatenate([c16v[1:], c16v[-1:]], axis=0)) * 0.5

    sc = jax.lax.dot_general(q, cmpk, (((1,), (1,)), ((), ())),
                             preferred_element_type=jnp.float32) * scale
    sc3 = sc.reshape(BQ, H, nch)
    trow = t0 + jax.lax.broadcasted_iota(jnp.int32, (BQ, 1, 1), 0)
    cidx = jax.lax.broadcasted_iota(jnp.int32, (1, 1, nch), 2)
    cmask = (cidx * STRIDE + (KS - 1)) <= trow
    p3 = _masked_softmax(sc3, cmask)
    cmp_o = jnp.dot(p3.reshape(BQH, nch), cmpv,
                    preferred_element_type=jnp.float32)

    p_sum = jnp.sum(p3, axis=1)
    p_slc = jnp.dot(p_sum, m_ref[...],
                    preferred_element_type=jnp.float32)

    tq = t0 + jax.lax.broadcasted_iota(jnp.int32, (BQ, 1), 0)
    jidx = jax.lax.broadcasted_iota(jnp.int32, (1, J), 1)
    blk_valid = (jidx * BS) <= tq
    cur = tq // BS
    forced = ((jidx < NINIT) | (jidx == cur)) & blk_valid
    score = jnp.where(blk_valid, p_slc + forced.astype(jnp.float32) * 1024.0, -1.0)

    JJ = J * J
    a = jnp.dot(score, e_ref[...][:, :JJ], precision=jax.lax.Precision.HIGHEST,
                preferred_element_type=jnp.float32)
    bb = jnp.dot(score, t_ref[...], precision=jax.lax.Precision.HIGHEST,
                 preferred_element_type=jnp.float32)
    pidx = jax.lax.broadcasted_iota(jnp.int32, (1, JJ), 1)
    lane_lt = (pidx % J) < (pidx // J)
    beats = (bb > a) | ((bb == a) & lane_lt)
    rank = jax.lax.dot_general(beats.astype(jnp.float32), e_ref[...][:, :JJ],
                               (((1,), (1,)), ((), ())),
                               preferred_element_type=jnp.float32)
    sel = (rank < min(TOPN, J)) & blk_valid

    selx = jnp.dot(sel.astype(jnp.float32), e_ref[...],
                   preferred_element_type=jnp.float32)

    sfull = jax.lax.dot_general(q, ks, (((1,), (1,)), ((), ())),
                                preferred_element_type=jnp.float32) * scale
    s3 = sfull.reshape(BQ, H, TS)
    es = jnp.exp(s3)
    causal_f = c_ref[0][:, None, :]
    winm_f = wm_ref[0][:, None, :]
    selm_f = selx[:, None, :] * causal_f
    e_slc = es * selm_f
    e_swa = es * winm_f
    den_slc = jnp.sum(e_slc, axis=-1, keepdims=True)
    den_swa = jnp.sum(e_swa, axis=-1, keepdims=True)
    slc_p = e_slc / den_slc
    swa_p = e_swa / den_swa
    slc_o = jnp.dot(slc_p.reshape(BQH, TS), vs, preferred_element_type=jnp.float32)
    swa_o = jnp.dot(swa_p.reshape(BQH, TS), vs, preferred_element_type=jnp.float32)

    g = jax.nn.sigmoid(w_ref[0])
    out = g[:, 0:1] * cmp_o + g[:, 1:2] * slc_o + g[:, 2:3] * swa_o
    o_ref[...] = out.reshape(1, BQ, H, D)


def kernel(q, k, v, combine_weight, cu_seqlens):
    T, H, D = q.shape
    nseq = cu_seqlens.shape[0] - 1
    TS = T // nseq
    BQ = 256
    J = (TS + BS - 1) // BS
    nch = TS // STRIDE

    C = (TS - KS) // STRIDE + 1
    M_np = np.zeros((nch, J), np.float32)
    for c in range(C):
        s0 = (c * STRIDE) // BS
        s1 = (c * STRIDE + KS - 1) // BS
        M_np[c, s0:s1 + 1] = 1.0
    E_np = np.zeros((J, TS), np.float32)
    for j in range(J):
        E_np[j, j * BS:(j + 1) * BS] = 1.0
    JJ = J * J
    TILE_np = np.zeros((J, JJ), np.float32)
    for jp in range(J):
        TILE_np[jp, jp::J] = 1.0
    NQB = TS // BQ
    tpos = np.arange(TS)
    caus_np = np.zeros((NQB, BQ, TS), np.float32)
    win_np = np.zeros((NQB, BQ, TS), np.float32)
    for i in range(NQB):
        t = (i * BQ + np.arange(BQ))[:, None]
        caus_np[i] = (tpos[None, :] <= t).astype(np.float32)
        win_np[i] = ((tpos[None, :] <= t) & (tpos[None, :] > t - WIN)).astype(np.float32)

    q4 = q.reshape(nseq, TS, H, D)
    k4 = k.reshape(nseq, TS, D)
    v4 = v.reshape(nseq, TS, D)
    w4 = combine_weight.reshape(nseq, TS * H, 3)

    fn = functools.partial(_nsa_kernel, BQ=BQ, TS=TS, H=H, D=D, J=J)
    out = pl.pallas_call(
        fn,
        grid=(nseq, TS // BQ),
        in_specs=[
            pl.BlockSpec((1, BQ, H, D), lambda b, i: (b, i, 0, 0)),
            pl.BlockSpec((1, TS, D), lambda b, i: (b, 0, 0)),
            pl.BlockSpec((1, TS, D), lambda b, i: (b, 0, 0)),
            pl.BlockSpec((1, BQ * H, 3), lambda b, i: (b, i, 0)),
            pl.BlockSpec((nch, J), lambda b, i: (0, 0)),
            pl.BlockSpec((J, TS), lambda b, i: (0, 0)),
            pl.BlockSpec((J, JJ), lambda b, i: (0, 0)),
            pl.BlockSpec((1, BQ, TS), lambda b, i: (i, 0, 0)),
            pl.BlockSpec((1, BQ, TS), lambda b, i: (i, 0, 0)),
        ],
        out_specs=pl.BlockSpec((1, BQ, H, D), lambda b, i: (b, i, 0, 0)),
        out_shape=jax.ShapeDtypeStruct((nseq, TS, H, D), jnp.float32),
    )(q4, k4, v4, w4, jnp.asarray(M_np), jnp.asarray(E_np),
      jnp.asarray(TILE_np), jnp.asarray(caus_np), jnp.asarray(win_np))
    return out.reshape(T, H, D)


# speedup vs baseline: 1.5069x; 1.0742x over previous
import functools

import numpy as np
import jax
import jax.numpy as jnp
from jax.experimental import pallas as pl
from jax.experimental.pallas import tpu as pltpu

KS = 32
STRIDE = 16
BS = 32
TOPN = 16
NINIT = 2
WIN = 512
NEG = -1e30


def _masked_softmax(s, mask):
    sm = jnp.where(mask, s, NEG)
    m = jnp.max(sm, axis=-1, keepdims=True)
    e = jnp.where(mask, jnp.exp(sm - m), 0.0)
    den = jnp.maximum(jnp.sum(e, axis=-1, keepdims=True), 1e-30)
    return e / den


def _nsa_kernel(q_ref, k_ref, v_ref, w_ref, m_ref, e_ref, t_ref, a_ref, c_ref,
                wm_ref, o_ref, *, BQ, TSK, H, D, J, IOFF):
    i = pl.program_id(1)
    t0 = (i + IOFF) * BQ
    BQH = BQ * H
    scale = D ** -0.5

    q = q_ref[0].reshape(BQH, D)
    ks = k_ref[0]
    vs = v_ref[0]

    nch = TSK // STRIDE
    c16k = jnp.mean(ks.reshape(nch, STRIDE, D), axis=1)
    c16v = jnp.mean(vs.reshape(nch, STRIDE, D), axis=1)
    cmpk = (c16k + jnp.concatenate([c16k[1:], c16k[-1:]], axis=0)) * 0.5
    cmpv = (c16v + jnp.concatenate([c16v[1:], c16v[-1:]], axis=0)) * 0.5

    sc = jax.lax.dot_general(q, cmpk, (((1,), (1,)), ((), ())),
                             preferred_element_type=jnp.float32) * scale
    sc3 = sc.reshape(BQ, H, nch)
    trow = t0 + jax.lax.broadcasted_iota(jnp.int32, (BQ, 1, 1), 0)
    cidx = jax.lax.broadcasted_iota(jnp.int32, (1, 1, nch), 2)
    cmask = (cidx * STRIDE + (KS - 1)) <= trow
    p3 = _masked_softmax(sc3, cmask)
    cmp_o = jnp.dot(p3.reshape(BQH, nch), cmpv,
                    preferred_element_type=jnp.float32)

    p_sum = jnp.sum(p3, axis=1)
    p_slc = jnp.dot(p_sum, m_ref[...],
                    preferred_element_type=jnp.float32)

    tq = t0 + jax.lax.broadcasted_iota(jnp.int32, (BQ, 1), 0)
    jidx = jax.lax.broadcasted_iota(jnp.int32, (1, J), 1)
    blk_valid = (jidx * BS) <= tq
    cur = tq // BS
    forced = ((jidx < NINIT) | (jidx == cur)) & blk_valid
    score = jnp.where(blk_valid, p_slc + forced.astype(jnp.float32) * 1024.0, -1.0)

    JJ = J * J
    a = jnp.dot(score, a_ref[...], precision=jax.lax.Precision.HIGHEST,
                preferred_element_type=jnp.float32)
    bb = jnp.dot(score, t_ref[...], precision=jax.lax.Precision.HIGHEST,
                 preferred_element_type=jnp.float32)
    pidx = jax.lax.broadcasted_iota(jnp.int32, (1, JJ), 1)
    lane_lt = (pidx % J) < (pidx // J)
    beats = (bb > a) | ((bb == a) & lane_lt)
    rank = jax.lax.dot_general(beats.astype(jnp.float32), a_ref[...],
                               (((1,), (1,)), ((), ())),
                               preferred_element_type=jnp.float32)
    sel = (rank < min(TOPN, J)) & blk_valid

    selx = jnp.dot(sel.astype(jnp.float32), e_ref[...],
                   preferred_element_type=jnp.float32)

    sfull = jax.lax.dot_general(q, ks, (((1,), (1,)), ((), ())),
                                preferred_element_type=jnp.float32) * scale
    s3 = sfull.reshape(BQ, H, TSK)
    es = jnp.exp(s3)
    causal_f = c_ref[0][:, None, :]
    winm_f = wm_ref[0][:, None, :]
    selm_f = selx[:, None, :] * causal_f
    e_slc = es * selm_f
    e_swa = es * winm_f
    den_slc = jnp.sum(e_slc, axis=-1, keepdims=True)
    den_swa = jnp.sum(e_swa, axis=-1, keepdims=True)
    slc_p = e_slc / den_slc
    swa_p = e_swa / den_swa
    slc_o = jnp.dot(slc_p.reshape(BQH, TSK), vs, preferred_element_type=jnp.float32)
    swa_o = jnp.dot(swa_p.reshape(BQH, TSK), vs, preferred_element_type=jnp.float32)

    g = jax.nn.sigmoid(w_ref[0])
    out = g[:, 0:1] * cmp_o + g[:, 1:2] * slc_o + g[:, 2:3] * swa_o
    o_ref[...] = out.reshape(1, BQ, H, D)


def kernel(q, k, v, combine_weight, cu_seqlens):
    T, H, D = q.shape
    nseq = cu_seqlens.shape[0] - 1
    TS = T // nseq
    BQ = 256
    J = (TS + BS - 1) // BS
    nch = TS // STRIDE

    C = (TS - KS) // STRIDE + 1
    M_np = np.zeros((nch, J), np.float32)
    for c in range(C):
        s0 = (c * STRIDE) // BS
        s1 = (c * STRIDE + KS - 1) // BS
        M_np[c, s0:s1 + 1] = 1.0
    E_np = np.zeros((J, TS), np.float32)
    for j in range(J):
        E_np[j, j * BS:(j + 1) * BS] = 1.0
    JJ = J * J
    EXPA_np = np.zeros((J, JJ), np.float32)
    TILE_np = np.zeros((J, JJ), np.float32)
    for j in range(J):
        EXPA_np[j, j * J:(j + 1) * J] = 1.0
        TILE_np[j, j::J] = 1.0

    q4 = q.reshape(nseq, TS, H, D)
    k4 = k.reshape(nseq, TS, D)
    v4 = v.reshape(nseq, TS, D)
    w4 = combine_weight.reshape(nseq, TS * H, 3)

    def run_half(ioff, nblk, tsk):
        nck = tsk // STRIDE
        tpos = np.arange(tsk)
        caus_np = np.zeros((nblk, BQ, tsk), np.float32)
        win_np = np.zeros((nblk, BQ, tsk), np.float32)
        for i in range(nblk):
            t = ((ioff + i) * BQ + np.arange(BQ))[:, None]
            caus_np[i] = (tpos[None, :] <= t).astype(np.float32)
            win_np[i] = ((tpos[None, :] <= t) &
                         (tpos[None, :] > t - WIN)).astype(np.float32)
        fn = functools.partial(_nsa_kernel, BQ=BQ, TSK=tsk, H=H, D=D, J=J,
                               IOFF=ioff)
        return pl.pallas_call(
            fn,
            grid=(nseq, nblk),
            in_specs=[
                pl.BlockSpec((1, BQ, H, D), lambda b, i: (b, i + ioff, 0, 0)),
                pl.BlockSpec((1, tsk, D), lambda b, i: (b, 0, 0)),
                pl.BlockSpec((1, tsk, D), lambda b, i: (b, 0, 0)),
                pl.BlockSpec((1, BQ * H, 3), lambda b, i: (b, i + ioff, 0)),
                pl.BlockSpec((nck, J), lambda b, i: (0, 0)),
                pl.BlockSpec((J, tsk), lambda b, i: (0, 0)),
                pl.BlockSpec((J, JJ), lambda b, i: (0, 0)),
                pl.BlockSpec((J, JJ), lambda b, i: (0, 0)),
                pl.BlockSpec((1, BQ, tsk), lambda b, i: (i, 0, 0)),
                pl.BlockSpec((1, BQ, tsk), lambda b, i: (i, 0, 0)),
            ],
            out_specs=pl.BlockSpec((1, BQ, H, D), lambda b, i: (b, i, 0, 0)),
            out_shape=jax.ShapeDtypeStruct((nseq, nblk * BQ, H, D), jnp.float32),
        )(q4, k4, v4, w4, jnp.asarray(M_np[:nck]), jnp.asarray(E_np[:, :tsk]),
          jnp.asarray(TILE_np), jnp.asarray(EXPA_np),
          jnp.asarray(caus_np), jnp.asarray(win_np))

    NQB = TS // BQ
    lo = run_half(0, NQB // 2, TS // 2)
    hi = run_half(NQB // 2, NQB - NQB // 2, TS)
    out = jnp.concatenate([lo, hi], axis=1)
    return out.reshape(T, H, D)


# speedup vs baseline: 1.5821x; 1.0500x over previous
import functools

import numpy as np
import jax
import jax.numpy as jnp
from jax.experimental import pallas as pl
from jax.experimental.pallas import tpu as pltpu

KS = 32
STRIDE = 16
BS = 32
TOPN = 16
NINIT = 2
WIN = 512
NEG = -1e30


def _masked_softmax(s, mask):
    sm = jnp.where(mask, s, NEG)
    m = jnp.max(sm, axis=-1, keepdims=True)
    e = jnp.where(mask, jnp.exp(sm - m), 0.0)
    den = jnp.maximum(jnp.sum(e, axis=-1, keepdims=True), 1e-30)
    return e / den


def _nsa_kernel(q_ref, k_ref, v_ref, w_ref, m_ref, e_ref, t_ref, a_ref, c_ref,
                wm_ref, o_ref, *, BQ, TSK, H, D, J, IOFF):
    i = pl.program_id(1)
    t0 = (i + IOFF) * BQ
    BQH = BQ * H
    scale = D ** -0.5

    q = q_ref[0].reshape(BQH, D)
    ks = k_ref[0]
    vs = v_ref[0]

    nch = TSK // STRIDE
    c16k = jnp.mean(ks.reshape(nch, STRIDE, D), axis=1)
    c16v = jnp.mean(vs.reshape(nch, STRIDE, D), axis=1)
    cmpk = (c16k + jnp.concatenate([c16k[1:], c16k[-1:]], axis=0)) * 0.5
    cmpv = (c16v + jnp.concatenate([c16v[1:], c16v[-1:]], axis=0)) * 0.5

    sc = jax.lax.dot_general(q, cmpk, (((1,), (1,)), ((), ())),
                             preferred_element_type=jnp.float32) * scale
    sc3 = sc.reshape(BQ, H, nch)
    trow = t0 + jax.lax.broadcasted_iota(jnp.int32, (BQ, 1, 1), 0)
    cidx = jax.lax.broadcasted_iota(jnp.int32, (1, 1, nch), 2)
    cmask = (cidx * STRIDE + (KS - 1)) <= trow
    p3 = _masked_softmax(sc3, cmask)
    cmp_o = jnp.dot(p3.reshape(BQH, nch), cmpv,
                    preferred_element_type=jnp.float32)

    p_sum = jnp.sum(p3, axis=1)
    p_slc = jnp.dot(p_sum, m_ref[...],
                    preferred_element_type=jnp.float32)

    tq = t0 + jax.lax.broadcasted_iota(jnp.int32, (BQ, 1), 0)
    jidx = jax.lax.broadcasted_iota(jnp.int32, (1, J), 1)
    blk_valid = (jidx * BS) <= tq
    cur = tq // BS
    forced = ((jidx < NINIT) | (jidx == cur)) & blk_valid
    score = jnp.where(blk_valid, p_slc + forced.astype(jnp.float32) * 1024.0, -1.0)

    JJ = J * J
    a = jnp.dot(score, a_ref[...], precision=jax.lax.Precision.HIGHEST,
                preferred_element_type=jnp.float32)
    bb = jnp.dot(score, t_ref[...], precision=jax.lax.Precision.HIGHEST,
                 preferred_element_type=jnp.float32)
    pidx = jax.lax.broadcasted_iota(jnp.int32, (1, JJ), 1)
    lane_lt = (pidx % J) < (pidx // J)
    beats = (bb > a) | ((bb == a) & lane_lt)
    rank = jax.lax.dot_general(beats.astype(jnp.float32), a_ref[...],
                               (((1,), (1,)), ((), ())),
                               preferred_element_type=jnp.float32)
    sel = (rank < min(TOPN, J)) & blk_valid

    selx = jnp.dot(sel.astype(jnp.float32), e_ref[...],
                   preferred_element_type=jnp.float32)

    sfull = jax.lax.dot_general(q, ks, (((1,), (1,)), ((), ())),
                                preferred_element_type=jnp.float32) * scale
    s3 = sfull.reshape(BQ, H, TSK)
    es = jnp.exp(s3)
    causal_f = c_ref[0][:, None, :]
    winm_f = wm_ref[0][:, None, :]
    selm_f = selx[:, None, :] * causal_f
    e_slc = es * selm_f
    e_swa = es * winm_f
    den_slc = jnp.sum(e_slc, axis=-1, keepdims=True)
    den_swa = jnp.sum(e_swa, axis=-1, keepdims=True)
    slc_p = e_slc / den_slc
    swa_p = e_swa / den_swa
    slc_o = jnp.dot(slc_p.reshape(BQH, TSK), vs, preferred_element_type=jnp.float32)
    swa_o = jnp.dot(swa_p.reshape(BQH, TSK), vs, preferred_element_type=jnp.float32)

    g = jax.nn.sigmoid(w_ref[0])
    out = g[:, 0:1] * cmp_o + g[:, 1:2] * slc_o + g[:, 2:3] * swa_o
    o_ref[...] = out.reshape(1, BQ, H, D)


def kernel(q, k, v, combine_weight, cu_seqlens):
    T, H, D = q.shape
    nseq = cu_seqlens.shape[0] - 1
    TS = T // nseq
    BQ = 256
    J = (TS + BS - 1) // BS
    nch = TS // STRIDE

    C = (TS - KS) // STRIDE + 1
    M_np = np.zeros((nch, J), np.float32)
    for c in range(C):
        s0 = (c * STRIDE) // BS
        s1 = (c * STRIDE + KS - 1) // BS
        M_np[c, s0:s1 + 1] = 1.0
    E_np = np.zeros((J, TS), np.float32)
    for j in range(J):
        E_np[j, j * BS:(j + 1) * BS] = 1.0
    JJ = J * J
    EXPA_np = np.zeros((J, JJ), np.float32)
    TILE_np = np.zeros((J, JJ), np.float32)
    for j in range(J):
        EXPA_np[j, j * J:(j + 1) * J] = 1.0
        TILE_np[j, j::J] = 1.0

    q4 = q.reshape(nseq, TS, H, D)
    k4 = k.reshape(nseq, TS, D)
    v4 = v.reshape(nseq, TS, D)
    w4 = combine_weight.reshape(nseq, TS * H, 3)

    def run_half(ioff, nblk, tsk):
        nck = tsk // STRIDE
        tpos = np.arange(tsk)
        caus_np = np.zeros((nblk, BQ, tsk), np.float32)
        win_np = np.zeros((nblk, BQ, tsk), np.float32)
        for i in range(nblk):
            t = ((ioff + i) * BQ + np.arange(BQ))[:, None]
            caus_np[i] = (tpos[None, :] <= t).astype(np.float32)
            win_np[i] = ((tpos[None, :] <= t) &
                         (tpos[None, :] > t - WIN)).astype(np.float32)
        fn = functools.partial(_nsa_kernel, BQ=BQ, TSK=tsk, H=H, D=D, J=J,
                               IOFF=ioff)
        return pl.pallas_call(
            fn,
            grid=(nseq, nblk),
            in_specs=[
                pl.BlockSpec((1, BQ, H, D), lambda b, i: (b, i + ioff, 0, 0)),
                pl.BlockSpec((1, tsk, D), lambda b, i: (b, 0, 0)),
                pl.BlockSpec((1, tsk, D), lambda b, i: (b, 0, 0)),
                pl.BlockSpec((1, BQ * H, 3), lambda b, i: (b, i + ioff, 0)),
                pl.BlockSpec((nck, J), lambda b, i: (0, 0)),
                pl.BlockSpec((J, tsk), lambda b, i: (0, 0)),
                pl.BlockSpec((J, JJ), lambda b, i: (0, 0)),
                pl.BlockSpec((J, JJ), lambda b, i: (0, 0)),
                pl.BlockSpec((1, BQ, tsk), lambda b, i: (i, 0, 0)),
                pl.BlockSpec((1, BQ, tsk), lambda b, i: (i, 0, 0)),
            ],
            out_specs=pl.BlockSpec((1, BQ, H, D), lambda b, i: (b, i, 0, 0)),
            out_shape=jax.ShapeDtypeStruct((nseq, nblk * BQ, H, D), jnp.float32),
        )(q4, k4, v4, w4, jnp.asarray(M_np[:nck]), jnp.asarray(E_np[:, :tsk]),
          jnp.asarray(TILE_np), jnp.asarray(EXPA_np),
          jnp.asarray(caus_np), jnp.asarray(win_np))

    NQB = TS // BQ
    parts = [run_half(i, 1, (i + 1) * BQ) for i in range(NQB)]
    out = jnp.concatenate(parts, axis=1)
    return out.reshape(T, H, D)
